# baseline (device time: 59779 ns/iter reference)
import jax
import jax.numpy as jnp
from jax import lax
from jax.experimental import pallas as pl
from jax.experimental.pallas import tpu as pltpu


def kernel(Q, K, V):
    B, S, H, D = Q.shape
    scale = D ** -0.5
    HD = H * D
    HALF = HD // 2

    def body(q_ref, k_ref, v_ref, out_ref, qbf_ref, kbuf_ref, vbuf_ref,
             ob_send_ref, ob_recv_ref,
             kv_send_sem, kv_recv_sem, out_send_sem, out_recv_sem):
        my_x = lax.axis_index("x")
        my_y = lax.axis_index("y")
        y_nbr = (my_x, 1 - my_y)
        x_nbr = (1 - my_x, my_y)

        barrier_sem = pltpu.get_barrier_semaphore()
        for nbr in (y_nbr, x_nbr):
            pl.semaphore_signal(
                barrier_sem, inc=1, device_id=nbr,
                device_id_type=pl.DeviceIdType.MESH,
            )
        pl.semaphore_wait(barrier_sem, 2)

        def run_half(h0):
            lo = h0 * D
            lanes = slice(lo, lo + HALF)
            oth0 = (H // 2) - h0
            olanes = slice(oth0 * D, oth0 * D + HALF)

            kbuf_ref[my_y, :, :, lanes] = k_ref[:, :, lanes].astype(jnp.bfloat16)
            vbuf_ref[my_y, :, :, lanes] = v_ref[:, :, lanes].astype(jnp.bfloat16)

            kv_rdmas = []
            for b in range(B):
                kr = pltpu.make_async_remote_copy(
                    src_ref=kbuf_ref.at[my_y, b, :, lanes],
                    dst_ref=kbuf_ref.at[my_y, b, :, lanes],
                    send_sem=kv_send_sem.at[0, b],
                    recv_sem=kv_recv_sem.at[0, b],
                    device_id=y_nbr,
                    device_id_type=pl.DeviceIdType.MESH,
                )
                vr = pltpu.make_async_remote_copy(
                    src_ref=vbuf_ref.at[my_y, b, :, lanes],
                    dst_ref=vbuf_ref.at[my_y, b, :, lanes],
                    send_sem=kv_send_sem.at[1, b],
                    recv_sem=kv_recv_sem.at[1, b],
                    device_id=y_nbr,
                    device_id_type=pl.DeviceIdType.MESH,
                )
                kr.start()
                vr.start()
                kv_rdmas.append((kr, vr))

            qbf_ref[:, :, lanes] = (q_ref[:, :, lanes] * scale).astype(jnp.bfloat16)

            out_rdmas = []
            for b in range(B):
                kr, vr = kv_rdmas[b]
                kr.wait_recv()
                vr.wait_recv()
                for j in range(H // 2):
                    h = h0 + j
                    hs = slice(h * D, (h + 1) * D)
                    q = qbf_ref[b, :, hs]
                    s0 = lax.dot_general(
                        q, kbuf_ref[0, b, :, hs], (((1,), (1,)), ((), ())),
                        preferred_element_type=jnp.float32,
                    )
                    s1 = lax.dot_general(
                        q, kbuf_ref[1, b, :, hs], (((1,), (1,)), ((), ())),
                        preferred_element_type=jnp.float32,
                    )
                    p0 = jnp.exp(s0)
                    p1 = jnp.exp(s1)
                    denom = (jnp.sum(p0, axis=1, keepdims=True)
                             + jnp.sum(p1, axis=1, keepdims=True))
                    acc = lax.dot_general(
                        p0.astype(jnp.bfloat16), vbuf_ref[0, b, :, hs],
                        (((1,), (0,)), ((), ())),
                        preferred_element_type=jnp.float32,
                    ) + lax.dot_general(
                        p1.astype(jnp.bfloat16), vbuf_ref[1, b, :, hs],
                        (((1,), (0,)), ((), ())),
                        preferred_element_type=jnp.float32,
                    )
                    res = acc / denom
                    out_ref[b, :, hs] = res
                    ob_send_ref[b, :, j * D:(j + 1) * D] = res.astype(jnp.bfloat16)
                orr = pltpu.make_async_remote_copy(
                    src_ref=ob_send_ref.at[b],
                    dst_ref=ob_recv_ref.at[b],
                    send_sem=out_send_sem.at[b],
                    recv_sem=out_recv_sem.at[b],
                    device_id=x_nbr,
                    device_id_type=pl.DeviceIdType.MESH,
                )
                orr.start()
                out_rdmas.append(orr)

            for b in range(B):
                out_rdmas[b].wait_recv()
                out_ref[b, :, olanes] = ob_recv_ref[b].astype(jnp.float32)

            for b in range(B):
                kr, vr = kv_rdmas[b]
                kr.wait_send()
                vr.wait_send()
                out_rdmas[b].wait_send()

        @pl.when(my_x == 0)
        def _():
            run_half(0)

        @pl.when(my_x == 1)
        def _():
            run_half(H // 2)

    out = pl.pallas_call(
        body,
        out_shape=jax.ShapeDtypeStruct((B, S, HD), jnp.float32),
        in_specs=[
            pl.BlockSpec(memory_space=pltpu.VMEM),
            pl.BlockSpec(memory_space=pltpu.VMEM),
            pl.BlockSpec(memory_space=pltpu.VMEM),
        ],
        out_specs=pl.BlockSpec(memory_space=pltpu.VMEM),
        scratch_shapes=[
            pltpu.VMEM((B, S, HD), jnp.bfloat16),
            pltpu.VMEM((2, B, S, HD), jnp.bfloat16),
            pltpu.VMEM((2, B, S, HD), jnp.bfloat16),
            pltpu.VMEM((B, S, HALF), jnp.bfloat16),
            pltpu.VMEM((B, S, HALF), jnp.bfloat16),
            pltpu.SemaphoreType.DMA((2, B)),
            pltpu.SemaphoreType.DMA((2, B)),
            pltpu.SemaphoreType.DMA((B,)),
            pltpu.SemaphoreType.DMA((B,)),
        ],
        compiler_params=pltpu.CompilerParams(
            collective_id=0,
            vmem_limit_bytes=100 * 1024 * 1024,
        ),
    )(Q.reshape(B, S, HD), K.reshape(B, S, HD), V.reshape(B, S, HD))
    return out.reshape(B, S, H, D)


# device time: 59715 ns/iter; 1.0011x vs baseline; 1.0011x over previous
import jax
import jax.numpy as jnp
from jax import lax
from jax.experimental import pallas as pl
from jax.experimental.pallas import tpu as pltpu


def kernel(Q, K, V):
    B, S, H, D = Q.shape
    scale = D ** -0.5
    HD = H * D
    HALF = HD // 2

    def body(q_ref, k_ref, v_ref, out_ref, qbf_ref, kbuf_ref, vbuf_ref,
             ob_send_ref, ob_recv_ref,
             kv_send_sem, kv_recv_sem, out_send_sem, out_recv_sem):
        my_x = lax.axis_index("x")
        my_y = lax.axis_index("y")
        y_nbr = (my_x, 1 - my_y)
        x_nbr = (1 - my_x, my_y)

        barrier_sem = pltpu.get_barrier_semaphore()
        for nbr in (y_nbr, x_nbr):
            pl.semaphore_signal(
                barrier_sem, inc=1, device_id=nbr,
                device_id_type=pl.DeviceIdType.MESH,
            )
        pl.semaphore_wait(barrier_sem, 2)

        def run_half(h0):
            lo = h0 * D
            lanes = slice(lo, lo + HALF)
            oth0 = (H // 2) - h0
            olanes = slice(oth0 * D, oth0 * D + HALF)

            kbuf_ref[my_y] = k_ref[:, :, lanes].astype(jnp.bfloat16)
            vbuf_ref[my_y] = v_ref[:, :, lanes].astype(jnp.bfloat16)

            kv_rdmas = []
            for b in range(B):
                kr = pltpu.make_async_remote_copy(
                    src_ref=kbuf_ref.at[my_y, b],
                    dst_ref=kbuf_ref.at[my_y, b],
                    send_sem=kv_send_sem.at[0, b],
                    recv_sem=kv_recv_sem.at[0, b],
                    device_id=y_nbr,
                    device_id_type=pl.DeviceIdType.MESH,
                )
                vr = pltpu.make_async_remote_copy(
                    src_ref=vbuf_ref.at[my_y, b],
                    dst_ref=vbuf_ref.at[my_y, b],
                    send_sem=kv_send_sem.at[1, b],
                    recv_sem=kv_recv_sem.at[1, b],
                    device_id=y_nbr,
                    device_id_type=pl.DeviceIdType.MESH,
                )
                kr.start()
                vr.start()
                kv_rdmas.append((kr, vr))

            qbf_ref[...] = (q_ref[:, :, lanes] * scale).astype(jnp.bfloat16)

            out_rdmas = []
            for b in range(B):
                kr, vr = kv_rdmas[b]
                kr.wait_recv()
                vr.wait_recv()
                for j in range(H // 2):
                    h = h0 + j
                    hs = slice(h * D, (h + 1) * D)
                    js = slice(j * D, (j + 1) * D)
                    q = qbf_ref[b, :, js]
                    s0 = lax.dot_general(
                        q, kbuf_ref[0, b, :, js], (((1,), (1,)), ((), ())),
                        preferred_element_type=jnp.float32,
                    )
                    s1 = lax.dot_general(
                        q, kbuf_ref[1, b, :, js], (((1,), (1,)), ((), ())),
                        preferred_element_type=jnp.float32,
                    )
                    p0 = jnp.exp(s0)
                    p1 = jnp.exp(s1)
                    denom = (jnp.sum(p0, axis=1, keepdims=True)
                             + jnp.sum(p1, axis=1, keepdims=True))
                    acc = lax.dot_general(
                        p0.astype(jnp.bfloat16), vbuf_ref[0, b, :, js],
                        (((1,), (0,)), ((), ())),
                        preferred_element_type=jnp.float32,
                    ) + lax.dot_general(
                        p1.astype(jnp.bfloat16), vbuf_ref[1, b, :, js],
                        (((1,), (0,)), ((), ())),
                        preferred_element_type=jnp.float32,
                    )
                    res = acc / denom
                    out_ref[b, :, hs] = res
                    ob_send_ref[b, :, js] = res.astype(jnp.bfloat16)
                orr = pltpu.make_async_remote_copy(
                    src_ref=ob_send_ref.at[b],
                    dst_ref=ob_recv_ref.at[b],
                    send_sem=out_send_sem.at[b],
                    recv_sem=out_recv_sem.at[b],
                    device_id=x_nbr,
                    device_id_type=pl.DeviceIdType.MESH,
                )
                orr.start()
                out_rdmas.append(orr)

            for b in range(B):
                out_rdmas[b].wait_recv()
                out_ref[b, :, olanes] = ob_recv_ref[b].astype(jnp.float32)

            for b in range(B):
                kr, vr = kv_rdmas[b]
                kr.wait_send()
                vr.wait_send()
                out_rdmas[b].wait_send()

        @pl.when(my_x == 0)
        def _():
            run_half(0)

        @pl.when(my_x == 1)
        def _():
            run_half(H // 2)

    out = pl.pallas_call(
        body,
        out_shape=jax.ShapeDtypeStruct((B, S, HD), jnp.float32),
        in_specs=[
            pl.BlockSpec(memory_space=pltpu.VMEM),
            pl.BlockSpec(memory_space=pltpu.VMEM),
            pl.BlockSpec(memory_space=pltpu.VMEM),
        ],
        out_specs=pl.BlockSpec(memory_space=pltpu.VMEM),
        scratch_shapes=[
            pltpu.VMEM((B, S, HALF), jnp.bfloat16),
            pltpu.VMEM((2, B, S, HALF), jnp.bfloat16),
            pltpu.VMEM((2, B, S, HALF), jnp.bfloat16),
            pltpu.VMEM((B, S, HALF), jnp.bfloat16),
            pltpu.VMEM((B, S, HALF), jnp.bfloat16),
            pltpu.SemaphoreType.DMA((2, B)),
            pltpu.SemaphoreType.DMA((2, B)),
            pltpu.SemaphoreType.DMA((B,)),
            pltpu.SemaphoreType.DMA((B,)),
        ],
        compiler_params=pltpu.CompilerParams(
            collective_id=0,
            vmem_limit_bytes=100 * 1024 * 1024,
        ),
    )(Q.reshape(B, S, HD), K.reshape(B, S, HD), V.reshape(B, S, HD))
    return out.reshape(B, S, H, D)


# device time: 54893 ns/iter; 1.0890x vs baseline; 1.0878x over previous
import jax
import jax.numpy as jnp
from jax import lax
from jax.experimental import pallas as pl
from jax.experimental.pallas import tpu as pltpu


def kernel(Q, K, V):
    B, S, H, D = Q.shape
    scale = D ** -0.5
    HD = H * D
    HALF = HD // 2

    def body(q_hbm, k_hbm, v_hbm, out_hbm,
             qf32_ref, kf32_ref, vf32_ref, oacc_ref,
             qbf_ref, kbuf_ref, vbuf_ref, ob_send_ref, ob_recv_ref,
             in_sem, out_dma_sem,
             kv_send_sem, kv_recv_sem, out_send_sem, out_recv_sem):
        my_x = lax.axis_index("x")
        my_y = lax.axis_index("y")
        y_nbr = (my_x, 1 - my_y)
        x_nbr = (1 - my_x, my_y)

        q_dmas, k_dmas, v_dmas = [], [], []
        for b in range(B):
            for lst, src, dst, row in (
                (k_dmas, k_hbm, kf32_ref, 1),
                (v_dmas, v_hbm, vf32_ref, 2),
                (q_dmas, q_hbm, qf32_ref, 0),
            ):
                dma = pltpu.make_async_copy(
                    src.at[b], dst.at[b], in_sem.at[row, b]
                )
                dma.start()
                lst.append(dma)

        barrier_sem = pltpu.get_barrier_semaphore()
        for nbr in (y_nbr, x_nbr):
            pl.semaphore_signal(
                barrier_sem, inc=1, device_id=nbr,
                device_id_type=pl.DeviceIdType.MESH,
            )
        pl.semaphore_wait(barrier_sem, 2)

        def run_half(h0):
            lo = h0 * D
            lanes = slice(lo, lo + HALF)
            oth0 = (H // 2) - h0
            olanes = slice(oth0 * D, oth0 * D + HALF)

            kv_rdmas = []
            for b in range(B):
                k_dmas[b].wait()
                kbuf_ref[my_y, b] = kf32_ref[b, :, lanes].astype(jnp.bfloat16)
                kr = pltpu.make_async_remote_copy(
                    src_ref=kbuf_ref.at[my_y, b],
                    dst_ref=kbuf_ref.at[my_y, b],
                    send_sem=kv_send_sem.at[0, b],
                    recv_sem=kv_recv_sem.at[0, b],
                    device_id=y_nbr,
                    device_id_type=pl.DeviceIdType.MESH,
                )
                kr.start()
                v_dmas[b].wait()
                vbuf_ref[my_y, b] = vf32_ref[b, :, lanes].astype(jnp.bfloat16)
                vr = pltpu.make_async_remote_copy(
                    src_ref=vbuf_ref.at[my_y, b],
                    dst_ref=vbuf_ref.at[my_y, b],
                    send_sem=kv_send_sem.at[1, b],
                    recv_sem=kv_recv_sem.at[1, b],
                    device_id=y_nbr,
                    device_id_type=pl.DeviceIdType.MESH,
                )
                vr.start()
                kv_rdmas.append((kr, vr))

            out_rdmas = []
            for b in range(B):
                q_dmas[b].wait()
                qbf_ref[b] = (qf32_ref[b, :, lanes] * scale).astype(jnp.bfloat16)
                kr, vr = kv_rdmas[b]
                kr.wait_recv()
                vr.wait_recv()
                for j in range(H // 2):
                    h = h0 + j
                    hs = slice(h * D, (h + 1) * D)
                    js = slice(j * D, (j + 1) * D)
                    q = qbf_ref[b, :, js]
                    s0 = lax.dot_general(
                        q, kbuf_ref[0, b, :, js], (((1,), (1,)), ((), ())),
                        preferred_element_type=jnp.float32,
                    )
                    s1 = lax.dot_general(
                        q, kbuf_ref[1, b, :, js], (((1,), (1,)), ((), ())),
                        preferred_element_type=jnp.float32,
                    )
                    p0 = jnp.exp(s0)
                    p1 = jnp.exp(s1)
                    denom = (jnp.sum(p0, axis=1, keepdims=True)
                             + jnp.sum(p1, axis=1, keepdims=True))
                    acc = lax.dot_general(
                        p0.astype(jnp.bfloat16), vbuf_ref[0, b, :, js],
                        (((1,), (0,)), ((), ())),
                        preferred_element_type=jnp.float32,
                    ) + lax.dot_general(
                        p1.astype(jnp.bfloat16), vbuf_ref[1, b, :, js],
                        (((1,), (0,)), ((), ())),
                        preferred_element_type=jnp.float32,
                    )
                    res = acc / denom
                    oacc_ref[b, :, hs] = res
                    ob_send_ref[b, :, js] = res.astype(jnp.bfloat16)
                orr = pltpu.make_async_remote_copy(
                    src_ref=ob_send_ref.at[b],
                    dst_ref=ob_recv_ref.at[b],
                    send_sem=out_send_sem.at[b],
                    recv_sem=out_recv_sem.at[b],
                    device_id=x_nbr,
                    device_id_type=pl.DeviceIdType.MESH,
                )
                orr.start()
                out_rdmas.append(orr)

            out_dmas = []
            for b in range(B):
                out_rdmas[b].wait_recv()
                oacc_ref[b, :, olanes] = ob_recv_ref[b].astype(jnp.float32)
                od = pltpu.make_async_copy(
                    oacc_ref.at[b], out_hbm.at[b], out_dma_sem.at[b]
                )
                od.start()
                out_dmas.append(od)

            for b in range(B):
                out_dmas[b].wait()
                kr, vr = kv_rdmas[b]
                kr.wait_send()
                vr.wait_send()
                out_rdmas[b].wait_send()

        @pl.when(my_x == 0)
        def _():
            run_half(0)

        @pl.when(my_x == 1)
        def _():
            run_half(H // 2)

    out = pl.pallas_call(
        body,
        out_shape=jax.ShapeDtypeStruct((B, S, HD), jnp.float32),
        in_specs=[
            pl.BlockSpec(memory_space=pltpu.MemorySpace.HBM),
            pl.BlockSpec(memory_space=pltpu.MemorySpace.HBM),
            pl.BlockSpec(memory_space=pltpu.MemorySpace.HBM),
        ],
        out_specs=pl.BlockSpec(memory_space=pltpu.MemorySpace.HBM),
        scratch_shapes=[
            pltpu.VMEM((B, S, HD), jnp.float32),
            pltpu.VMEM((B, S, HD), jnp.float32),
            pltpu.VMEM((B, S, HD), jnp.float32),
            pltpu.VMEM((B, S, HD), jnp.float32),
            pltpu.VMEM((B, S, HALF), jnp.bfloat16),
            pltpu.VMEM((2, B, S, HALF), jnp.bfloat16),
            pltpu.VMEM((2, B, S, HALF), jnp.bfloat16),
            pltpu.VMEM((B, S, HALF), jnp.bfloat16),
            pltpu.VMEM((B, S, HALF), jnp.bfloat16),
            pltpu.SemaphoreType.DMA((3, B)),
            pltpu.SemaphoreType.DMA((B,)),
            pltpu.SemaphoreType.DMA((2, B)),
            pltpu.SemaphoreType.DMA((2, B)),
            pltpu.SemaphoreType.DMA((B,)),
            pltpu.SemaphoreType.DMA((B,)),
        ],
        compiler_params=pltpu.CompilerParams(
            collective_id=0,
            vmem_limit_bytes=100 * 1024 * 1024,
        ),
    )(Q.reshape(B, S, HD), K.reshape(B, S, HD), V.reshape(B, S, HD))
    return out.reshape(B, S, H, D)
